# baseline (speedup 1.0000x reference)
import functools

import jax
import jax.numpy as jnp
from jax.experimental import pallas as pl
from jax.experimental.pallas import tpu as pltpu

_B, _L, _D = 32, 1024, 768
_DV, _K, _NQ = 64, 512, 4
_N = _B * _L
_T = 1024
_W = 3 * _DV + 3


def _mm_bf16(a, b):
    return jax.lax.dot_general(
        a, b, (((1,), (0,)), ((), ())),
        preferred_element_type=jnp.float32)


def _rowsum64(v):
    s = v[:, 0:8]
    for g in range(1, 8):
        s = s + v[:, 8 * g:8 * (g + 1)]
    s = s[:, :4] + s[:, 4:]
    s = s[:, :2] + s[:, 2:]
    return s[:, 0:1] + s[:, 1:2]


def _rowsum64_t(x):
    v = jnp.transpose(x)
    v = v * v
    s = v[0:8]
    for g in range(1, 8):
        s = s + v[8 * g:8 * (g + 1)]
    s = s[0:4] + s[4:8]
    s = s[0:2] + s[2:4]
    s = s[0:1] + s[1:2]
    return jnp.transpose(s)


def _rvq_body(x_ref, win_ref, bin_ref, cb_ref, wout_ref, bout_ref,
              z_ref, idx_ref, loss_ref, rhs_ref, c2_ref):
    i = pl.program_id(0)

    @pl.when(i == 0)
    def _init():
        loss_ref[...] = jnp.zeros_like(loss_ref)
        kiota = jax.lax.broadcasted_iota(jnp.int32, (_K, 1), 0)
        aug2 = jnp.concatenate(
            [jnp.ones((_K, 1), jnp.float32),
             (kiota & 255).astype(jnp.float32),
             (kiota & 256).astype(jnp.float32)], axis=1).astype(jnp.bfloat16)
        for q in range(_NQ):
            cb = cb_ref[q]
            cb_p0 = cb.astype(jnp.bfloat16)
            cb_r = cb - cb_p0.astype(jnp.float32)
            cb_p1 = cb_r.astype(jnp.bfloat16)
            cb_p2 = (cb_r - cb_p1.astype(jnp.float32)).astype(jnp.bfloat16)
            rhs_ref[q] = jnp.concatenate([cb_p0, cb_p1, cb_p2, aug2], axis=1)
            c2_ref[q] = _rowsum64(cb * cb).reshape(1, _K)

    xb = x_ref[...]
    xp = jax.lax.dot_general(
        xb.astype(jnp.bfloat16), win_ref[...].astype(jnp.bfloat16),
        (((1,), (0,)), ((), ())),
        preferred_element_type=jnp.float32)
    res = xp + bin_ref[...]

    qsum = jnp.zeros_like(res)
    loss = jnp.float32(0.0)
    idx_cols = []
    for q in range(_NQ):
        rhs = rhs_ref[q]
        rc = jax.lax.dot_general(
            res.astype(jnp.bfloat16), rhs[:, 0:_DV],
            (((1,), (1,)), ((), ())),
            preferred_element_type=jnp.float32)
        r2 = _rowsum64_t(res)
        d2 = (r2 - 2.0 * rc) + c2_ref[q]
        m = jnp.min(d2, axis=1, keepdims=True)
        eqb = (d2 == m).astype(jnp.bfloat16)
        agg = _mm_bf16(eqb, rhs)
        cnt = agg[:, 3 * _DV:3 * _DV + 1]
        quant = ((agg[:, 0:_DV] + agg[:, _DV:2 * _DV])
                 + agg[:, 2 * _DV:3 * _DV])

        def _slow(d2=d2, m=m, rhs=rhs):
            lane_iota = jax.lax.broadcasted_iota(
                jnp.int32, (_T, _K), 1).astype(jnp.float32)
            idxf = jnp.min(jnp.where(d2 == m, lane_iota, jnp.float32(_K)),
                           axis=1, keepdims=True)
            onehot = (lane_iota == idxf).astype(jnp.bfloat16)
            s = _mm_bf16(onehot, rhs)
            quant = ((s[:, 0:_DV] + s[:, _DV:2 * _DV])
                     + s[:, 2 * _DV:3 * _DV])
            return idxf, quant

        def _fast(agg=agg, quant=quant):
            return (agg[:, 3 * _DV + 1:3 * _DV + 2]
                    + agg[:, 3 * _DV + 2:3 * _DV + 3]), quant

        idxf, quant = jax.lax.cond(
            jnp.max(cnt) > 1.5, _slow, _fast)
        diff = quant - res
        loss = loss + jnp.sum(diff * diff)
        qst = res + (quant - res)
        res = res - qst
        qsum = qsum + qst
        idx_cols.append(idxf.astype(jnp.int32))

    z = jax.lax.dot_general(
        qsum.astype(jnp.bfloat16), wout_ref[...].astype(jnp.bfloat16),
        (((1,), (0,)), ((), ())),
        preferred_element_type=jnp.float32)
    z_ref[...] = z + bout_ref[...]
    idx_ref[...] = jnp.concatenate(idx_cols, axis=1)
    acc = loss_ref[...] + jnp.reshape(loss, (1, 1))
    scale = jnp.where(i == pl.num_programs(0) - 1,
                      jnp.float32(1.0 / (_N * _DV)), jnp.float32(1.0))
    loss_ref[...] = acc * scale


@functools.partial(jax.jit, static_argnames=("interpret",))
def kernel(x, W_in, b_in, codebooks, W_out, b_out, interpret=False):
    xf = x.reshape(_N, _D)
    grid = (_N // _T,)
    z, idx, loss = pl.pallas_call(
        _rvq_body,
        grid=grid,
        in_specs=[
            pl.BlockSpec((_T, _D), lambda i: (i, 0)),
            pl.BlockSpec((_D, _DV), lambda i: (0, 0)),
            pl.BlockSpec((1, _DV), lambda i: (0, 0)),
            pl.BlockSpec((_NQ, _K, _DV), lambda i: (0, 0, 0)),
            pl.BlockSpec((_DV, _D), lambda i: (0, 0)),
            pl.BlockSpec((1, _D), lambda i: (0, 0)),
        ],
        out_specs=[
            pl.BlockSpec((_T, _D), lambda i: (i, 0)),
            pl.BlockSpec((_T, _NQ), lambda i: (i, 0)),
            pl.BlockSpec((1, 1), lambda i: (0, 0)),
        ],
        out_shape=[
            jax.ShapeDtypeStruct((_N, _D), jnp.float32),
            jax.ShapeDtypeStruct((_N, _NQ), jnp.int32),
            jax.ShapeDtypeStruct((1, 1), jnp.float32),
        ],
        scratch_shapes=[
            pltpu.VMEM((_NQ, _K, _W), jnp.bfloat16),
            pltpu.VMEM((_NQ, 1, _K), jnp.float32),
        ],
        interpret=interpret,
    )(xf, W_in, b_in.reshape(1, _DV), codebooks, W_out, b_out.reshape(1, _D))
    return (z.reshape(_B, _L, _D), idx.reshape(_B, _L, _NQ), loss[0, 0])


# speedup vs baseline: 2.2887x; 2.2887x over previous
import functools

import jax
import jax.numpy as jnp
from jax.experimental import pallas as pl
from jax.experimental.pallas import tpu as pltpu

_B, _L, _D = 32, 1024, 768
_DV, _K, _NQ = 64, 512, 4
_N = _B * _L
_T = 1024
_W = 3 * _DV + 3


def _mm_bf16(a, b, dims):
    return jax.lax.dot_general(a, b, (dims, ((), ())),
                               preferred_element_type=jnp.float32)


def _rowsum64(v):
    s = v[:, 0:8]
    for g in range(1, 8):
        s = s + v[:, 8 * g:8 * (g + 1)]
    s = s[:, :4] + s[:, 4:]
    s = s[:, :2] + s[:, 2:]
    return s[:, 0:1] + s[:, 1:2]


def _colsum64(v):
    s = v[0:8]
    for g in range(1, 8):
        s = s + v[8 * g:8 * (g + 1)]
    s = s[0:4] + s[4:8]
    s = s[0:2] + s[2:4]
    return s[0:1] + s[1:2]


def _rvq_body(x_ref, win_ref, bin_ref, cb_ref, wout_ref, bout_ref,
              z_ref, idx_ref, loss_ref, rhs_ref, c2_ref):
    i = pl.program_id(0)

    @pl.when(i == 0)
    def _init():
        loss_ref[...] = jnp.zeros_like(loss_ref)
        kiota = jax.lax.broadcasted_iota(jnp.int32, (_K, 1), 0)
        aug2 = jnp.concatenate(
            [jnp.ones((_K, 1), jnp.float32),
             (kiota & 255).astype(jnp.float32),
             (kiota & 256).astype(jnp.float32)], axis=1).astype(jnp.bfloat16)
        for q in range(_NQ):
            cb = cb_ref[q]
            cb_p0 = cb.astype(jnp.bfloat16)
            cb_r = cb - cb_p0.astype(jnp.float32)
            cb_p1 = cb_r.astype(jnp.bfloat16)
            cb_p2 = (cb_r - cb_p1.astype(jnp.float32)).astype(jnp.bfloat16)
            rhs_ref[q] = jnp.concatenate([cb_p0, cb_p1, cb_p2, aug2], axis=1)
            c2_ref[q] = _rowsum64(cb * cb)

    xb = x_ref[...]
    xpt = _mm_bf16(win_ref[...].astype(jnp.bfloat16),
                   xb.astype(jnp.bfloat16), ((0,), (1,)))
    rest = xpt + bin_ref[...]

    qsumt = jnp.zeros_like(rest)
    loss = jnp.float32(0.0)
    idx_rows = []
    for q in range(_NQ):
        rhs = rhs_ref[q]
        rct = _mm_bf16(rhs[:, 0:_DV], rest.astype(jnp.bfloat16),
                       ((1,), (0,)))
        r2t = _colsum64(rest * rest)
        d2t = (r2t - 2.0 * rct) + c2_ref[q]
        mt = jnp.min(d2t, axis=0, keepdims=True)
        eqbt = (d2t == mt).astype(jnp.bfloat16)
        aggt = _mm_bf16(rhs, eqbt, ((0,), (0,)))
        cnt = aggt[3 * _DV:3 * _DV + 1]
        quant = ((aggt[0:_DV] + aggt[_DV:2 * _DV])
                 + aggt[2 * _DV:3 * _DV])

        def _slow(d2t=d2t, mt=mt, rhs=rhs):
            sub_iota = jax.lax.broadcasted_iota(
                jnp.int32, (_K, _T), 0).astype(jnp.float32)
            idxf = jnp.min(jnp.where(d2t == mt, sub_iota, jnp.float32(_K)),
                           axis=0, keepdims=True)
            onehot = (sub_iota == idxf).astype(jnp.bfloat16)
            s = _mm_bf16(rhs, onehot, ((0,), (0,)))
            quant = ((s[0:_DV] + s[_DV:2 * _DV]) + s[2 * _DV:3 * _DV])
            return idxf, quant

        def _fast(aggt=aggt, quant=quant):
            return (aggt[3 * _DV + 1:3 * _DV + 2]
                    + aggt[3 * _DV + 2:3 * _DV + 3]), quant

        idxf, quant = jax.lax.cond(
            jnp.max(cnt) > 1.5, _slow, _fast)
        diff = quant - rest
        loss = loss + jnp.sum(diff * diff)
        qst = rest + (quant - rest)
        rest = rest - qst
        qsumt = qsumt + qst
        idx_rows.append(idxf.astype(jnp.int32))

    z = _mm_bf16(qsumt.astype(jnp.bfloat16),
                 wout_ref[...].astype(jnp.bfloat16), ((0,), (0,)))
    z_ref[...] = z + bout_ref[...]
    idx_ref[...] = jnp.concatenate(idx_rows, axis=0)
    acc = loss_ref[...] + jnp.reshape(loss, (1, 1))
    scale = jnp.where(i == pl.num_programs(0) - 1,
                      jnp.float32(1.0 / (_N * _DV)), jnp.float32(1.0))
    loss_ref[...] = acc * scale


@functools.partial(jax.jit, static_argnames=("interpret",))
def kernel(x, W_in, b_in, codebooks, W_out, b_out, interpret=False):
    xf = x.reshape(_N, _D)
    grid = (_N // _T,)
    z, idx, loss = pl.pallas_call(
        _rvq_body,
        grid=grid,
        in_specs=[
            pl.BlockSpec((_T, _D), lambda i: (i, 0)),
            pl.BlockSpec((_D, _DV), lambda i: (0, 0)),
            pl.BlockSpec((_DV, 1), lambda i: (0, 0)),
            pl.BlockSpec((_NQ, _K, _DV), lambda i: (0, 0, 0)),
            pl.BlockSpec((_DV, _D), lambda i: (0, 0)),
            pl.BlockSpec((1, _D), lambda i: (0, 0)),
        ],
        out_specs=[
            pl.BlockSpec((_T, _D), lambda i: (i, 0)),
            pl.BlockSpec((_NQ, _T), lambda i: (0, i)),
            pl.BlockSpec((1, 1), lambda i: (0, 0)),
        ],
        out_shape=[
            jax.ShapeDtypeStruct((_N, _D), jnp.float32),
            jax.ShapeDtypeStruct((_NQ, _N), jnp.int32),
            jax.ShapeDtypeStruct((1, 1), jnp.float32),
        ],
        scratch_shapes=[
            pltpu.VMEM((_NQ, _K, _W), jnp.bfloat16),
            pltpu.VMEM((_NQ, _K, 1), jnp.float32),
        ],
        interpret=interpret,
    )(xf, W_in, b_in.reshape(_DV, 1), codebooks, W_out, b_out.reshape(1, _D))
    return (z.reshape(_B, _L, _D),
            jnp.transpose(idx).reshape(_B, _L, _NQ),
            loss[0, 0])


# speedup vs baseline: 2.5682x; 1.1221x over previous
import functools

import jax
import jax.numpy as jnp
from jax.experimental import pallas as pl
from jax.experimental.pallas import tpu as pltpu

_B, _L, _D = 32, 1024, 768
_DV, _K, _NQ = 64, 512, 4
_N = _B * _L
_T = 2048
_W = 3 * _DV + 3


def _mm_bf16(a, b, dims):
    return jax.lax.dot_general(a, b, (dims, ((), ())),
                               preferred_element_type=jnp.float32)


def _rowsum64(v):
    s = v[:, 0:8]
    for g in range(1, 8):
        s = s + v[:, 8 * g:8 * (g + 1)]
    s = s[:, :4] + s[:, 4:]
    s = s[:, :2] + s[:, 2:]
    return s[:, 0:1] + s[:, 1:2]


def _colsum64(v):
    s = v[0:8]
    for g in range(1, 8):
        s = s + v[8 * g:8 * (g + 1)]
    s = s[0:4] + s[4:8]
    s = s[0:2] + s[2:4]
    return s[0:1] + s[1:2]


def _rvq_body(x_ref, win_ref, bin_ref, cb_ref, wout_ref, bout_ref,
              z_ref, idx_ref, loss_ref, rhs_ref, c2_ref):
    i = pl.program_id(0)

    @pl.when(i == 0)
    def _init():
        loss_ref[...] = jnp.zeros_like(loss_ref)
        kiota = jax.lax.broadcasted_iota(jnp.int32, (_K, 1), 0)
        aug2 = jnp.concatenate(
            [jnp.ones((_K, 1), jnp.float32),
             (kiota & 255).astype(jnp.float32),
             (kiota & 256).astype(jnp.float32)], axis=1).astype(jnp.bfloat16)
        for q in range(_NQ):
            cb = cb_ref[q]
            cb_p0 = cb.astype(jnp.bfloat16)
            cb_r = cb - cb_p0.astype(jnp.float32)
            cb_p1 = cb_r.astype(jnp.bfloat16)
            cb_p2 = (cb_r - cb_p1.astype(jnp.float32)).astype(jnp.bfloat16)
            rhs_ref[q] = jnp.concatenate([cb_p0, cb_p1, cb_p2, aug2], axis=1)
            c2_ref[q] = _rowsum64(cb * cb)

    xb = x_ref[...]
    xpt = _mm_bf16(win_ref[...].astype(jnp.bfloat16),
                   xb.astype(jnp.bfloat16), ((0,), (1,)))
    rest = xpt + bin_ref[...]

    qsumt = jnp.zeros_like(rest)
    loss = jnp.float32(0.0)
    idx_rows = []
    for q in range(_NQ):
        rhs = rhs_ref[q]
        rct = _mm_bf16(rhs[:, 0:_DV], rest.astype(jnp.bfloat16),
                       ((1,), (0,)))
        r2t = _colsum64(rest * rest)
        d2t = (r2t - 2.0 * rct) + c2_ref[q]
        mt = jnp.min(d2t, axis=0, keepdims=True)
        eqbt = (d2t == mt).astype(jnp.bfloat16)
        aggt = _mm_bf16(rhs, eqbt, ((0,), (0,)))
        cnt = aggt[3 * _DV:3 * _DV + 1]
        quant = ((aggt[0:_DV] + aggt[_DV:2 * _DV])
                 + aggt[2 * _DV:3 * _DV])

        def _slow(d2t=d2t, mt=mt, rhs=rhs):
            sub_iota = jax.lax.broadcasted_iota(
                jnp.int32, (_K, _T), 0).astype(jnp.float32)
            idxf = jnp.min(jnp.where(d2t == mt, sub_iota, jnp.float32(_K)),
                           axis=0, keepdims=True)
            onehot = (sub_iota == idxf).astype(jnp.bfloat16)
            s = _mm_bf16(rhs, onehot, ((0,), (0,)))
            quant = ((s[0:_DV] + s[_DV:2 * _DV]) + s[2 * _DV:3 * _DV])
            return idxf, quant

        def _fast(aggt=aggt, quant=quant):
            return (aggt[3 * _DV + 1:3 * _DV + 2]
                    + aggt[3 * _DV + 2:3 * _DV + 3]), quant

        idxf, quant = jax.lax.cond(
            jnp.max(cnt) > 1.5, _slow, _fast)
        diff = quant - rest
        loss = loss + jnp.sum(diff * diff)
        qst = rest + (quant - rest)
        rest = rest - qst
        qsumt = qsumt + qst
        idx_rows.append(idxf.astype(jnp.int32))

    z = _mm_bf16(qsumt.astype(jnp.bfloat16),
                 wout_ref[...].astype(jnp.bfloat16), ((0,), (0,)))
    z_ref[...] = z + bout_ref[...]
    idx_ref[...] = jnp.concatenate(idx_rows, axis=0)
    acc = loss_ref[...] + jnp.reshape(loss, (1, 1))
    scale = jnp.where(i == pl.num_programs(0) - 1,
                      jnp.float32(1.0 / (_N * _DV)), jnp.float32(1.0))
    loss_ref[...] = acc * scale


@functools.partial(jax.jit, static_argnames=("interpret",))
def kernel(x, W_in, b_in, codebooks, W_out, b_out, interpret=False):
    xf = x.reshape(_N, _D)
    grid = (_N // _T,)
    z, idx, loss = pl.pallas_call(
        _rvq_body,
        grid=grid,
        in_specs=[
            pl.BlockSpec((_T, _D), lambda i: (i, 0)),
            pl.BlockSpec((_D, _DV), lambda i: (0, 0)),
            pl.BlockSpec((_DV, 1), lambda i: (0, 0)),
            pl.BlockSpec((_NQ, _K, _DV), lambda i: (0, 0, 0)),
            pl.BlockSpec((_DV, _D), lambda i: (0, 0)),
            pl.BlockSpec((1, _D), lambda i: (0, 0)),
        ],
        out_specs=[
            pl.BlockSpec((_T, _D), lambda i: (i, 0)),
            pl.BlockSpec((_NQ, _T), lambda i: (0, i)),
            pl.BlockSpec((1, 1), lambda i: (0, 0)),
        ],
        out_shape=[
            jax.ShapeDtypeStruct((_N, _D), jnp.float32),
            jax.ShapeDtypeStruct((_NQ, _N), jnp.int32),
            jax.ShapeDtypeStruct((1, 1), jnp.float32),
        ],
        scratch_shapes=[
            pltpu.VMEM((_NQ, _K, _W), jnp.bfloat16),
            pltpu.VMEM((_NQ, _K, 1), jnp.float32),
        ],
        interpret=interpret,
    )(xf, W_in, b_in.reshape(_DV, 1), codebooks, W_out, b_out.reshape(1, _D))
    return (z.reshape(_B, _L, _D),
            jnp.transpose(idx).reshape(_B, _L, _NQ),
            loss[0, 0])
